# Initial kernel scaffold; baseline (speedup 1.0000x reference)
import jax
import jax.numpy as jnp
from jax.experimental import pallas as pl


def kernel(x, adjs, sparse, W_gcn, b_gcn, W_sem, b_sem, q_sem):
    raise NotImplementedError("write your pallas kernel here")


# speedup vs baseline: 1.0036x; 1.0036x over previous
import jax
import jax.numpy as jnp
from jax.experimental import pallas as pl

_BM = 200


def _proj_body(x_ref, wg_ref, h_ref, *, p_total):
    xb = x_ref[...].astype(jnp.bfloat16)
    for p in range(p_total):
        h = jnp.dot(xb, wg_ref[p].astype(jnp.bfloat16),
                    preferred_element_type=jnp.float32)
        h_ref[p] = h.astype(jnp.bfloat16)


def _agg_body(adj_ref, h_ref, bgcn_ref, wsem_ref, bsem_ref, qsem_ref,
              hrelu_ref, att_ref, *, p_total):
    for p in range(p_total):
        a = adj_ref[p].astype(jnp.bfloat16)
        acc = jnp.dot(a, h_ref[p], preferred_element_type=jnp.float32)
        acc = acc + bgcn_ref[p:p + 1, :]
        hr = jnp.maximum(acc, 0.0)
        hrelu_ref[p] = hr
        t = jnp.tanh(jnp.dot(hr, wsem_ref[...],
                             preferred_element_type=jnp.float32)
                     + bsem_ref[...])
        s = jnp.sum(t * qsem_ref[...])
        att_ref[p, 0] = jnp.full((8, 128), s, jnp.float32)


def _combine_body(hrelu_ref, att_ref, out_ref, *, p_total, n_rows):
    logits = [jnp.sum(att_ref[p]) * (1.0 / (1024.0 * n_rows))
              for p in range(p_total)]
    m = logits[0]
    for p in range(1, p_total):
        m = jnp.maximum(m, logits[p])
    exps = [jnp.exp(l - m) for l in logits]
    denom = exps[0]
    for p in range(1, p_total):
        denom = denom + exps[p]
    out = (exps[0] / denom) * hrelu_ref[0]
    for p in range(1, p_total):
        out = out + (exps[p] / denom) * hrelu_ref[p]
    out_ref[0] = out


def kernel(x, adjs, sparse, W_gcn, b_gcn, W_sem, b_sem, q_sem):
    import functools

    p_total, n, _ = adjs.shape
    nhid = W_gcn.shape[2]
    mblks = n // _BM

    h = pl.pallas_call(
        functools.partial(_proj_body, p_total=p_total),
        out_shape=jax.ShapeDtypeStruct((p_total, n, nhid), jnp.bfloat16),
    )(x, W_gcn)

    hrelu, att_part = pl.pallas_call(
        functools.partial(_agg_body, p_total=p_total),
        grid=(mblks,),
        in_specs=[
            pl.BlockSpec((p_total, _BM, n), lambda m: (0, m, 0)),
            pl.BlockSpec((p_total, n, nhid), lambda m: (0, 0, 0)),
            pl.BlockSpec(b_gcn.shape, lambda m: (0, 0)),
            pl.BlockSpec(W_sem.shape, lambda m: (0, 0)),
            pl.BlockSpec(b_sem.shape, lambda m: (0, 0)),
            pl.BlockSpec(q_sem.shape, lambda m: (0, 0)),
        ],
        out_specs=[
            pl.BlockSpec((p_total, _BM, nhid), lambda m: (0, m, 0)),
            pl.BlockSpec((p_total, 1, 8, 128), lambda m: (0, m, 0, 0)),
        ],
        out_shape=[
            jax.ShapeDtypeStruct((p_total, n, nhid), jnp.float32),
            jax.ShapeDtypeStruct((p_total, mblks, 8, 128), jnp.float32),
        ],
    )(adjs, h, b_gcn, W_sem, b_sem, q_sem)

    out = pl.pallas_call(
        functools.partial(_combine_body, p_total=p_total, n_rows=n),
        out_shape=jax.ShapeDtypeStruct((1, n, nhid), jnp.float32),
    )(hrelu, att_part)
    return out
